# baseline (speedup 1.0000x reference)
import functools

import jax
import jax.numpy as jnp
from jax import lax
from jax.experimental import pallas as pl
from jax.experimental.pallas import tpu as pltpu
from jax.experimental.pallas import tpu_sc as plsc

NUM_CLASSES = 100000
FEAT_DIM = 64
BATCH = 16384

NC = 2
NS = 16
L = 16
NW = NC * NS
DPW = FEAT_DIM // NW
BCHUNK = 4096
NBCH = BATCH // BCHUNK
UNROLL = 4
NSPLIT = 4
SEG_OFF = (0, 25088, 50048, 75136)
SEG_SZ = (25088, 24960, 25088, 24864)


def _body(feat_hbm, lab_hbm, cent_hbm, out_hbm, t0_v, t1_v, t2_v, t3_v,
          lab_v, f_v, acc_v, tsem0, tsem1, tsem2, tsem3, lsem, fsem0, fsem1):
    c = lax.axis_index("c")
    s = lax.axis_index("s")
    wid = s * NC + c
    fsems = (fsem0, fsem1)
    tbufs = (t0_v, t1_v, t2_v, t3_v)
    tsems = (tsem0, tsem1, tsem2, tsem3)

    seq = [(d, sg, ch) for d in range(DPW) for sg in range(NSPLIT)
           for ch in range(NBCH)]

    def issue_f(k):
        d, _, ch = seq[k]
        return pltpu.async_copy(
            feat_hbm.at[wid * DPW + d, pl.ds(ch * BCHUNK, BCHUNK)],
            f_v.at[k % 2], fsems[k % 2])

    def issue_t(d, sg):
        return pltpu.async_copy(
            cent_hbm.at[wid * DPW + d, pl.ds(SEG_OFF[sg], SEG_SZ[sg])],
            tbufs[sg], tsems[sg])

    tdesc = {sg: issue_t(0, sg) for sg in range(NSPLIT)}
    lb = pltpu.async_copy(lab_hbm, lab_v, lsem)
    fdesc = {0: issue_f(0), 1: issue_f(1)}
    lb.wait()

    zero = jnp.zeros((L,), jnp.float32)
    accs = (zero,) * UNROLL

    for k, (d, sg, ch) in enumerate(seq):
        if ch == 0:
            tdesc[sg].wait()
        fdesc[k].wait()
        cbase = ch * BCHUNK
        fb = k % 2
        lo = SEG_OFF[sg]

        def group(g, carry):
            out = []
            for u in range(UNROLL):
                off = (g * UNROLL + u) * L
                idx = lab_v[pl.ds(cbase + off, L)]
                fv = f_v[fb, pl.ds(off, L)]
                if sg == 0:
                    m = idx < SEG_SZ[0]
                    tv = plsc.load_gather(tbufs[sg], [idx], mask=m)
                else:
                    m = (idx >= lo) & (idx < lo + SEG_SZ[sg])
                    tv = plsc.load_gather(tbufs[sg], [idx - lo], mask=m)
                dd = fv - tv
                out.append(carry[u] + jnp.where(m, dd * dd, 0.0))
            return tuple(out)

        accs = lax.fori_loop(0, BCHUNK // (L * UNROLL), group, accs)

        if k + 2 < len(seq):
            fdesc[k + 2] = issue_f(k + 2)
        if d == 0 and ch == NBCH - 1 and DPW > 1:
            tdesc[sg] = issue_t(1, sg)

    acc_v[...] = (accs[0] + accs[1]) + (accs[2] + accs[3])
    pltpu.sync_copy(acc_v, out_hbm.at[wid])


_partials = functools.partial(
    pl.kernel,
    out_type=jax.ShapeDtypeStruct((NW, L), jnp.float32),
    mesh=plsc.VectorSubcoreMesh(core_axis_name="c", subcore_axis_name="s",
                                num_cores=NC, num_subcores=NS),
    scratch_types=[
        pltpu.VMEM((SEG_SZ[0],), jnp.float32),
        pltpu.VMEM((SEG_SZ[1],), jnp.float32),
        pltpu.VMEM((SEG_SZ[2],), jnp.float32),
        pltpu.VMEM((SEG_SZ[3],), jnp.float32),
        pltpu.VMEM((BATCH,), jnp.int32),
        pltpu.VMEM((2, BCHUNK), jnp.float32),
        pltpu.VMEM((L,), jnp.float32),
        pltpu.SemaphoreType.DMA,
        pltpu.SemaphoreType.DMA,
        pltpu.SemaphoreType.DMA,
        pltpu.SemaphoreType.DMA,
        pltpu.SemaphoreType.DMA,
        pltpu.SemaphoreType.DMA,
        pltpu.SemaphoreType.DMA,
    ],
    compiler_params=pltpu.CompilerParams(needs_layout_passes=False),
)(_body)


@jax.jit
def kernel(features, labels, centers):
    batch_size = features.shape[0]
    partials = _partials(features.T, labels.astype(jnp.int32), centers.T)
    return jnp.sum(partials) / (2.0 * batch_size)


# speedup vs baseline: 1.2733x; 1.2733x over previous
import functools

import jax
import jax.numpy as jnp
from jax import lax
from jax.experimental import pallas as pl
from jax.experimental.pallas import tpu as pltpu
from jax.experimental.pallas import tpu_sc as plsc

NUM_CLASSES = 100000
FEAT_DIM = 64
BATCH = 16384

NC = 2
NS = 16
L = 16
NW = NC * NS
DPW = FEAT_DIM // NW
BCHUNK = 4096
NBCH = BATCH // BCHUNK
UNROLL = 4
HALF_A = 50048
HALF_B = NUM_CLASSES - HALF_A


def _body(feat_hbm, lab_hbm, cent_hbm, out_hbm, ta_v, tb_v, lab_v, f_v,
          acc_v, tsema, tsemb, lsem, fsem0, fsem1):
    c = lax.axis_index("c")
    s = lax.axis_index("s")
    wid = s * NC + c
    fsems = (fsem0, fsem1)

    seq = [(d, p, ch) for d in range(DPW) for p in range(2)
           for ch in range(NBCH)]

    def issue_f(k):
        d, _, ch = seq[k]
        return pltpu.async_copy(
            feat_hbm.at[wid * DPW + d, pl.ds(ch * BCHUNK, BCHUNK)],
            f_v.at[k % 2], fsems[k % 2])

    def issue_ta(d):
        return pltpu.async_copy(cent_hbm.at[wid * DPW + d, pl.ds(0, HALF_A)],
                                ta_v, tsema)

    def issue_tb(d):
        return pltpu.async_copy(
            cent_hbm.at[wid * DPW + d, pl.ds(HALF_A, HALF_B)], tb_v, tsemb)

    tda = issue_ta(0)
    lb = pltpu.async_copy(lab_hbm, lab_v, lsem)
    fdesc = {0: issue_f(0), 1: issue_f(1)}
    tdb = issue_tb(0)
    lb.wait()

    zero = jnp.zeros((L,), jnp.float32)
    accs = (zero,) * UNROLL

    for k, (d, p, ch) in enumerate(seq):
        if p == 0 and ch == 0:
            tda.wait()
        if p == 1 and ch == 0:
            tdb.wait()
        fdesc[k].wait()
        cbase = ch * BCHUNK
        fb = k % 2

        def group(g, carry):
            out = []
            for u in range(UNROLL):
                off = (g * UNROLL + u) * L
                idx = lab_v[pl.ds(cbase + off, L)]
                fv = f_v[fb, pl.ds(off, L)]
                if p == 0:
                    m = idx < HALF_A
                    tv = plsc.load_gather(ta_v, [idx], mask=m)
                else:
                    m = idx >= HALF_A
                    tv = plsc.load_gather(tb_v, [idx - HALF_A], mask=m)
                dd = fv - tv
                out.append(carry[u] + jnp.where(m, dd * dd, 0.0))
            return tuple(out)

        accs = lax.fori_loop(0, BCHUNK // (L * UNROLL), group, accs)

        if k + 2 < len(seq):
            fdesc[k + 2] = issue_f(k + 2)
        if d == 0 and p == 0 and ch == NBCH - 1 and DPW > 1:
            tda = issue_ta(1)
        if d == 0 and p == 1 and ch == NBCH - 1 and DPW > 1:
            tdb = issue_tb(1)

    acc_v[...] = (accs[0] + accs[1]) + (accs[2] + accs[3])
    pltpu.sync_copy(acc_v, out_hbm.at[wid])


_partials = functools.partial(
    pl.kernel,
    out_type=jax.ShapeDtypeStruct((NW, L), jnp.float32),
    mesh=plsc.VectorSubcoreMesh(core_axis_name="c", subcore_axis_name="s",
                                num_cores=NC, num_subcores=NS),
    scratch_types=[
        pltpu.VMEM((HALF_A,), jnp.float32),
        pltpu.VMEM((HALF_B,), jnp.float32),
        pltpu.VMEM((BATCH,), jnp.int32),
        pltpu.VMEM((2, BCHUNK), jnp.float32),
        pltpu.VMEM((L,), jnp.float32),
        pltpu.SemaphoreType.DMA,
        pltpu.SemaphoreType.DMA,
        pltpu.SemaphoreType.DMA,
        pltpu.SemaphoreType.DMA,
        pltpu.SemaphoreType.DMA,
    ],
    compiler_params=pltpu.CompilerParams(needs_layout_passes=False),
)(_body)


@jax.jit
def kernel(features, labels, centers):
    batch_size = features.shape[0]
    partials = _partials(features.T, labels.astype(jnp.int32), centers.T)
    return jnp.sum(partials) / (2.0 * batch_size)
